# Initial kernel scaffold; baseline (speedup 1.0000x reference)
import jax
import jax.numpy as jnp
from jax.experimental import pallas as pl


def kernel(h, adj, inputs, ew1, eb1, ew2, eb2, bnw, bnb, eps1, w1, b1, w2, b2, wg, bg, wl, bl):
    raise NotImplementedError("write your pallas kernel here")


# speedup vs baseline: 1.1407x; 1.1407x over previous
import functools

import jax
import jax.numpy as jnp
from jax.experimental import pallas as pl
from jax.experimental.pallas import tpu as pltpu

N = 10000
D_IN = 128
HID = 64
ENC = 64
B = 4
T = 12

ADJ_BLK = 400
ROW_BLK = 2000


def _x_kernel(h_ref, w_ref, b_ref, o_ref):
    h = h_ref[...]
    w = w_ref[...]
    o_ref[...] = jnp.dot(h, w, preferred_element_type=jnp.float32) + b_ref[...]


def _adj_kernel(adj_ref, x_ref, eps_ref, x2_ref, stats_ref):
    i = pl.program_id(0)
    a = adj_ref[...]
    x = x_ref[...]
    pooled = jax.lax.dot_general(
        a.astype(jnp.bfloat16), x.astype(jnp.bfloat16),
        (((1,), (0,)), ((), ())),
        preferred_element_type=jnp.float32)
    degree = jnp.sum(a, axis=1, keepdims=True)
    degree = jnp.where(degree < 1e-6, jnp.float32(1.0), degree)
    xb = x_ref[pl.ds(i * ADJ_BLK, ADJ_BLK), :]
    x2 = pooled / degree + eps_ref[0] * xb
    x2_ref[...] = x2

    @pl.when(i == 0)
    def _init():
        stats_ref[...] = jnp.zeros_like(stats_ref)

    s = jnp.sum(x2, axis=0, keepdims=True)
    sq = jnp.sum(x2 * x2, axis=0, keepdims=True)
    stats_ref[pl.ds(0, 1), :] += s
    stats_ref[pl.ds(1, 1), :] += sq


def _feat_kernel(x2_ref, stats_ref, bnw_ref, bnb_ref, w2t_ref, eb2_ref,
                 wg_ref, bg_ref, wlt_ref, bl_ref, o_ref):
    mean = stats_ref[pl.ds(0, 1), :] / N
    var = stats_ref[pl.ds(1, 1), :] / N - mean * mean
    rstd = jax.lax.rsqrt(var + 1e-5)
    xn = (x2_ref[...] - mean) * rstd * bnw_ref[...] + bnb_ref[...]
    feat = jnp.dot(xn, w2t_ref[...], preferred_element_type=jnp.float32) \
        + eb2_ref[...]
    ft = jnp.dot(feat, wg_ref[...], preferred_element_type=jnp.float32) \
        + bg_ref[0]
    o_ref[...] = jnp.dot(ft, wlt_ref[...],
                         preferred_element_type=jnp.float32) + bl_ref[...]


def _gate_kernel(h_ref, inp_ref, w1h_ref, w1r_ref, b1_ref, a_ref):
    h = h_ref[...]
    z = jnp.dot(h.astype(jnp.bfloat16), w1h_ref[...].astype(jnp.bfloat16),
                preferred_element_type=jnp.float32)
    z = z + inp_ref[...] * w1r_ref[...] + b1_ref[...]
    a_ref[...] = jax.nn.sigmoid(z)


def _step_kernel(r_ref, u_ref, h_ref, inp_ref, ftc_ref,
                 w2h_ref, w2r_ref, b2_ref, wlt_ref, o_ref):
    h = h_ref[...]
    rh = r_ref[...] * h
    z = jnp.dot(rh.astype(jnp.bfloat16), w2h_ref[...].astype(jnp.bfloat16),
                preferred_element_type=jnp.float32)
    c = jnp.tanh(z + inp_ref[...] * w2r_ref[...] + b2_ref[...])
    u = u_ref[...]
    nh = u * h + (1.0 - u) * c
    o_ref[...] = jnp.dot(nh.astype(jnp.bfloat16),
                         wlt_ref[...].astype(jnp.bfloat16),
                         preferred_element_type=jnp.float32) + ftc_ref[...]


@jax.jit
def kernel(h, adj, inputs, ew1, eb1, ew2, eb2, bnw, bnb, eps1,
           w1, b1, w2, b2, wg, bg, wl, bl):
    f32 = jnp.float32

    x = pl.pallas_call(
        _x_kernel,
        out_shape=jax.ShapeDtypeStruct((N, HID), f32),
        in_specs=[pl.BlockSpec((N, D_IN), lambda: (0, 0)),
                  pl.BlockSpec((D_IN, HID), lambda: (0, 0)),
                  pl.BlockSpec((1, HID), lambda: (0, 0))],
        out_specs=pl.BlockSpec((N, HID), lambda: (0, 0)),
    )(h, ew1.T, eb1[None, :])

    nblk = N // ADJ_BLK
    x2, stats = pl.pallas_call(
        _adj_kernel,
        grid=(nblk,),
        out_shape=(jax.ShapeDtypeStruct((N, HID), f32),
                   jax.ShapeDtypeStruct((8, HID), f32)),
        in_specs=[pl.BlockSpec((ADJ_BLK, N), lambda i: (i, 0)),
                  pl.BlockSpec((N, HID), lambda i: (0, 0)),
                  pl.BlockSpec(memory_space=pltpu.SMEM)],
        out_specs=(pl.BlockSpec((ADJ_BLK, HID), lambda i: (i, 0)),
                   pl.BlockSpec((8, HID), lambda i: (0, 0))),
    )(adj, x, eps1)

    ftc = pl.pallas_call(
        _feat_kernel,
        out_shape=jax.ShapeDtypeStruct((N, HID), f32),
        in_specs=[pl.BlockSpec((N, HID), lambda: (0, 0)),
                  pl.BlockSpec((8, HID), lambda: (0, 0)),
                  pl.BlockSpec((1, HID), lambda: (0, 0)),
                  pl.BlockSpec((1, HID), lambda: (0, 0)),
                  pl.BlockSpec((HID, ENC), lambda: (0, 0)),
                  pl.BlockSpec((1, ENC), lambda: (0, 0)),
                  pl.BlockSpec((ENC, ENC), lambda: (0, 0)),
                  pl.BlockSpec(memory_space=pltpu.SMEM),
                  pl.BlockSpec((ENC, HID), lambda: (0, 0)),
                  pl.BlockSpec((1, HID), lambda: (0, 0))],
        out_specs=pl.BlockSpec((N, HID), lambda: (0, 0)),
    )(x2, stats, bnw[None, :], bnb[None, :], ew2.T, eb2[None, :],
      wg, bg, wl[:, HID:].T, bl[None, :])

    w1h = w1[1:, :]
    w1r = w1[0:1, :]
    w2h = w2[1:, :]
    w2r = w2[0:1, :]
    wlt = wl[:, :HID].T

    BN = B * N
    nrb = BN // ROW_BLK
    rb_per_b = N // ROW_BLK

    gate_call = pl.pallas_call(
        _gate_kernel,
        grid=(nrb,),
        out_shape=jax.ShapeDtypeStruct((BN, 2 * HID), f32),
        in_specs=[pl.BlockSpec((ROW_BLK, HID), lambda i: (i, 0)),
                  pl.BlockSpec((ROW_BLK, 1), lambda i: (i, 0)),
                  pl.BlockSpec((HID, 2 * HID), lambda i: (0, 0)),
                  pl.BlockSpec((1, 2 * HID), lambda i: (0, 0)),
                  pl.BlockSpec((1, 2 * HID), lambda i: (0, 0))],
        out_specs=pl.BlockSpec((ROW_BLK, 2 * HID), lambda i: (i, 0)),
    )

    step_call = pl.pallas_call(
        _step_kernel,
        grid=(nrb,),
        out_shape=jax.ShapeDtypeStruct((BN, HID), f32),
        in_specs=[pl.BlockSpec((ROW_BLK, HID), lambda i: (i, 0)),
                  pl.BlockSpec((ROW_BLK, HID), lambda i: (i, 0)),
                  pl.BlockSpec((ROW_BLK, HID), lambda i: (i, 0)),
                  pl.BlockSpec((ROW_BLK, 1), lambda i: (i, 0)),
                  pl.BlockSpec((ROW_BLK, HID), lambda i: (i % rb_per_b, 0)),
                  pl.BlockSpec((HID, HID), lambda i: (0, 0)),
                  pl.BlockSpec((1, HID), lambda i: (0, 0)),
                  pl.BlockSpec((1, HID), lambda i: (0, 0)),
                  pl.BlockSpec((HID, HID), lambda i: (0, 0)),
        ],
        out_specs=pl.BlockSpec((ROW_BLK, HID), lambda i: (i, 0)),
    )

    hidden = jnp.zeros((BN, HID), f32)
    for t in range(T):
        inp_t = inputs[:, t, :].reshape(BN, 1)
        a = gate_call(hidden, inp_t, w1h, w1r, b1[None, :])
        aflat = a.reshape(B, N * 2 * HID)
        r2d = aflat[:, :N * HID].reshape(BN, HID)
        u2d = aflat[:, N * HID:].reshape(BN, HID)
        hidden = step_call(r2d, u2d, hidden, inp_t, ftc,
                           w2h, w2r, b2[None, :], wlt)
    return hidden.reshape(B, N, HID)


# speedup vs baseline: 3.3154x; 2.9066x over previous
import jax
import jax.numpy as jnp
from jax.experimental import pallas as pl
from jax.experimental.pallas import tpu as pltpu

N = 10000
D_IN = 128
HID = 64
ENC = 64
B = 4
T = 12
BN = B * N

ADJ_BLK = 400
NADJ = N // ADJ_BLK

BLK = 2000
NBLK = BN // BLK
NB_PER_B = N // BLK
HB = BLK // 2


def _extract_kernel(adj_ref, h_ref, ew1t_ref, eb1_ref, eps_ref,
                    bnw_ref, bnb_ref, ew2t_ref, eb2_ref, wg_ref, bg_ref,
                    wlt2_ref, bl_ref, ftc_ref,
                    x_scr, xs_scr, x2_scr, stats_scr):
    i = pl.program_id(0)

    @pl.when(i == 0)
    def _init():
        x = jnp.dot(h_ref[...], ew1t_ref[...],
                    preferred_element_type=jnp.float32) + eb1_ref[...]
        x_scr[...] = x
        xs_scr[:, :HID] = x.astype(jnp.bfloat16)
        xs_scr[:, HID:HID + 1] = jnp.ones((N, 1), jnp.bfloat16)
        xs_scr[:, HID + 1:] = jnp.zeros((N, D_IN - HID - 1), jnp.bfloat16)
        stats_scr[...] = jnp.zeros_like(stats_scr)

    a = adj_ref[...]
    po = jnp.dot(a.astype(jnp.bfloat16), xs_scr[...],
                 preferred_element_type=jnp.float32)
    pooled = po[:, :HID]
    degree = po[:, HID:HID + 1]
    degree = jnp.where(degree < 1e-6, jnp.float32(1.0), degree)
    xb = x_scr[pl.ds(i * ADJ_BLK, ADJ_BLK), :]
    x2 = pooled / degree + eps_ref[0] * xb
    x2_scr[pl.ds(i * ADJ_BLK, ADJ_BLK), :] = x2
    stats_scr[0:1, :] += jnp.sum(x2, axis=0, keepdims=True)
    stats_scr[1:2, :] += jnp.sum(x2 * x2, axis=0, keepdims=True)

    @pl.when(i == NADJ - 1)
    def _feat():
        mean = stats_scr[0:1, :] / N
        var = stats_scr[1:2, :] / N - mean * mean
        rstd = jax.lax.rsqrt(var + 1e-5)
        xn = (x2_scr[...] - mean) * rstd * bnw_ref[...] + bnb_ref[...]
        feat = jnp.dot(xn, ew2t_ref[...],
                       preferred_element_type=jnp.float32) + eb2_ref[...]
        ft = jnp.dot(feat, wg_ref[...],
                     preferred_element_type=jnp.float32) + bg_ref[0]
        ftc_ref[...] = jnp.dot(ft, wlt2_ref[...],
                               preferred_element_type=jnp.float32) \
            + bl_ref[...]


def _gru_kernel(inp_ref, ftc_ref, w1h_ref, w1r_ref, b1_ref,
                w2h_ref, w2r_ref, b2_ref, wlt_ref, o_ref,
                a_scr, rg_scr, ug_scr):
    t = pl.program_id(0)
    ph = pl.program_id(1)
    blk = pl.program_id(2)
    g0 = blk * BLK

    @pl.when((t == 0) & (ph == 0) & (blk == 0))
    def _zero():
        o_ref[...] = jnp.zeros((BN, HID), jnp.float32)

    iv = inp_ref[...]
    lane = jax.lax.broadcasted_iota(jnp.int32, (BLK, T), 1)
    icol = jnp.sum(jnp.where(lane == t, iv, 0.0), axis=1,
                   keepdims=True)

    @pl.when(ph == 0)
    def _gates():
        h = o_ref[pl.ds(g0, BLK), :]
        z = jnp.dot(h.astype(jnp.bfloat16), w1h_ref[...].astype(jnp.bfloat16),
                    preferred_element_type=jnp.float32)
        z = z + icol * w1r_ref[...] + b1_ref[...]
        a_scr[pl.ds(g0, BLK), :] = jax.nn.sigmoid(z)

    @pl.when(ph == 1)
    def _update():
        b = blk // NB_PER_B
        j = blk % NB_PER_B
        r0 = b * N + j * HB
        u0 = r0 + N // 2
        ar = a_scr[pl.ds(r0, HB), :]
        au = a_scr[pl.ds(u0, HB), :]
        rg_scr[0::2, :] = ar[:, :HID]
        rg_scr[1::2, :] = ar[:, HID:]
        ug_scr[0::2, :] = au[:, :HID]
        ug_scr[1::2, :] = au[:, HID:]
        h = o_ref[pl.ds(g0, BLK), :]
        rh = rg_scr[...] * h
        z2 = jnp.dot(rh.astype(jnp.bfloat16),
                     w2h_ref[...].astype(jnp.bfloat16),
                     preferred_element_type=jnp.float32)
        c = jnp.tanh(z2 + icol * w2r_ref[...] + b2_ref[...])
        u = ug_scr[...]
        nh = u * h + (1.0 - u) * c
        hn = jnp.dot(nh.astype(jnp.bfloat16),
                     wlt_ref[...].astype(jnp.bfloat16),
                     preferred_element_type=jnp.float32) + ftc_ref[...]
        o_ref[pl.ds(g0, BLK), :] = hn


@jax.jit
def kernel(h, adj, inputs, ew1, eb1, ew2, eb2, bnw, bnb, eps1,
           w1, b1, w2, b2, wg, bg, wl, bl):
    f32 = jnp.float32
    const2 = lambda i: (0, 0)

    ftc = pl.pallas_call(
        _extract_kernel,
        grid=(NADJ,),
        out_shape=jax.ShapeDtypeStruct((N, HID), f32),
        in_specs=[pl.BlockSpec((ADJ_BLK, N), lambda i: (i, 0)),
                  pl.BlockSpec((N, D_IN), const2),
                  pl.BlockSpec((D_IN, HID), const2),
                  pl.BlockSpec((1, HID), const2),
                  pl.BlockSpec(memory_space=pltpu.SMEM),
                  pl.BlockSpec((1, HID), const2),
                  pl.BlockSpec((1, HID), const2),
                  pl.BlockSpec((HID, ENC), const2),
                  pl.BlockSpec((1, ENC), const2),
                  pl.BlockSpec((ENC, ENC), const2),
                  pl.BlockSpec(memory_space=pltpu.SMEM),
                  pl.BlockSpec((ENC, HID), const2),
                  pl.BlockSpec((1, HID), const2)],
        out_specs=pl.BlockSpec((N, HID), const2),
        scratch_shapes=[pltpu.VMEM((N, HID), f32),
                        pltpu.VMEM((N, D_IN), jnp.bfloat16),
                        pltpu.VMEM((N, HID), f32),
                        pltpu.VMEM((8, HID), f32)],
    )(adj, h, ew1.T, eb1[None, :], eps1, bnw[None, :], bnb[None, :],
      ew2.T, eb2[None, :], wg, bg, wl[:, HID:].T, bl[None, :])

    inp_cm = inputs.transpose(0, 2, 1).reshape(BN, T)

    out = pl.pallas_call(
        _gru_kernel,
        grid=(T, 2, NBLK),
        out_shape=jax.ShapeDtypeStruct((BN, HID), f32),
        in_specs=[pl.BlockSpec((BLK, T), lambda t, p, k: (k, 0)),
                  pl.BlockSpec((BLK, HID), lambda t, p, k: (k % NB_PER_B, 0)),
                  pl.BlockSpec((HID, 2 * HID), lambda t, p, k: (0, 0)),
                  pl.BlockSpec((1, 2 * HID), lambda t, p, k: (0, 0)),
                  pl.BlockSpec((1, 2 * HID), lambda t, p, k: (0, 0)),
                  pl.BlockSpec((HID, HID), lambda t, p, k: (0, 0)),
                  pl.BlockSpec((1, HID), lambda t, p, k: (0, 0)),
                  pl.BlockSpec((1, HID), lambda t, p, k: (0, 0)),
                  pl.BlockSpec((HID, HID), lambda t, p, k: (0, 0))],
        out_specs=pl.BlockSpec((BN, HID), lambda t, p, k: (0, 0)),
        scratch_shapes=[pltpu.VMEM((BN, 2 * HID), f32),
                        pltpu.VMEM((BLK, HID), f32),
                        pltpu.VMEM((BLK, HID), f32)],
    )(inp_cm, ftc, w1[1:, :], w1[0:1, :], b1[None, :],
      w2[1:, :], w2[0:1, :], b2[None, :], wl[:, :HID].T)

    return out.reshape(B, N, HID)


# speedup vs baseline: 5.2956x; 1.5973x over previous
import jax
import jax.numpy as jnp
from jax.experimental import pallas as pl
from jax.experimental.pallas import tpu as pltpu

N = 10000
D_IN = 128
HID = 64
ENC = 64
B = 4
T = 12
BN = B * N

ADJ_BLK = 400
NADJ = N // ADJ_BLK

BLK = 2000
NBLK = BN // BLK
NB_PER_B = N // BLK
HB = BLK // 2


def _extract_kernel(adj_ref, h_ref, ew1t_ref, eb1_ref, eps_ref,
                    bnw_ref, bnb_ref, ew2t_ref, eb2_ref, wg_ref, bg_ref,
                    wlt2_ref, bl_ref, ftc_ref,
                    x_scr, xs_scr, x2_scr, stats_scr):
    i = pl.program_id(0)

    @pl.when(i == 0)
    def _init():
        x = jnp.dot(h_ref[...], ew1t_ref[...],
                    preferred_element_type=jnp.float32) + eb1_ref[...]
        x_scr[...] = x
        xs_scr[:, :HID] = x.astype(jnp.bfloat16)
        xs_scr[:, HID:HID + 1] = jnp.ones((N, 1), jnp.bfloat16)
        xs_scr[:, HID + 1:] = jnp.zeros((N, D_IN - HID - 1), jnp.bfloat16)
        stats_scr[...] = jnp.zeros_like(stats_scr)

    a = adj_ref[...]
    po = jnp.dot(a.astype(jnp.bfloat16), xs_scr[...],
                 preferred_element_type=jnp.float32)
    pooled = po[:, :HID]
    degree = po[:, HID:HID + 1]
    degree = jnp.where(degree < 1e-6, jnp.float32(1.0), degree)
    xb = x_scr[pl.ds(i * ADJ_BLK, ADJ_BLK), :]
    x2 = pooled / degree + eps_ref[0] * xb
    x2_scr[pl.ds(i * ADJ_BLK, ADJ_BLK), :] = x2
    stats_scr[0:1, :] += jnp.sum(x2, axis=0, keepdims=True)
    stats_scr[1:2, :] += jnp.sum(x2 * x2, axis=0, keepdims=True)

    @pl.when(i == NADJ - 1)
    def _feat():
        mean = stats_scr[0:1, :] / N
        var = stats_scr[1:2, :] / N - mean * mean
        rstd = jax.lax.rsqrt(var + 1e-5)
        xn = (x2_scr[...] - mean) * rstd * bnw_ref[...] + bnb_ref[...]
        feat = jnp.dot(xn, ew2t_ref[...],
                       preferred_element_type=jnp.float32) + eb2_ref[...]
        ft = jnp.dot(feat, wg_ref[...],
                     preferred_element_type=jnp.float32) + bg_ref[0]
        ftc_ref[...] = jnp.dot(ft, wlt2_ref[...],
                               preferred_element_type=jnp.float32) \
            + bl_ref[...]


def _gru_kernel(inp_ref, ftc_ref, w1h_ref, w1r_ref, b1_ref,
                w2h_ref, w2r_ref, b2_ref, wlt_ref, o_ref,
                a_scr, rg_scr, ug_scr):
    t = pl.program_id(0)
    bf16 = jnp.bfloat16
    f32 = jnp.float32

    @pl.when(t == 0)
    def _zero():
        o_ref[...] = jnp.zeros((BN, HID), f32)

    ftcv = ftc_ref[...]
    for b in range(B):
        h_b = o_ref[pl.ds(b * N, N), :]
        ir = inp_ref[pl.ds(t, 1), pl.ds(b, 1), :].reshape(1, N)
        irb = ir.astype(bf16)
        z = jnp.dot(h_b.astype(bf16), w1h_ref[...].astype(bf16),
                    preferred_element_type=f32)
        z = z + jax.lax.dot_general(
            irb, w1r_ref[...].astype(bf16),
            (((0,), (0,)), ((), ())), preferred_element_type=f32)
        a_scr[...] = jax.nn.sigmoid(z + b1_ref[...]).astype(bf16)

        ar = a_scr[pl.ds(0, N // 2), :].astype(f32)
        au = a_scr[pl.ds(N // 2, N // 2), :].astype(f32)
        rg_scr[0::2, :] = ar[:, :HID]
        rg_scr[1::2, :] = ar[:, HID:]
        ug_scr[0::2, :] = au[:, :HID]
        ug_scr[1::2, :] = au[:, HID:]

        rh = rg_scr[...] * h_b
        z2 = jnp.dot(rh.astype(bf16), w2h_ref[...].astype(bf16),
                     preferred_element_type=f32)
        z2 = z2 + jax.lax.dot_general(
            irb, w2r_ref[...].astype(bf16),
            (((0,), (0,)), ((), ())), preferred_element_type=f32)
        c = jnp.tanh(z2 + b2_ref[...])
        u = ug_scr[...]
        nh = u * h_b + (1.0 - u) * c
        hn = jnp.dot(nh.astype(bf16), wlt_ref[...].astype(bf16),
                     preferred_element_type=f32)
        o_ref[pl.ds(b * N, N), :] = hn + ftcv


@jax.jit
def kernel(h, adj, inputs, ew1, eb1, ew2, eb2, bnw, bnb, eps1,
           w1, b1, w2, b2, wg, bg, wl, bl):
    f32 = jnp.float32
    const2 = lambda i: (0, 0)

    ftc = pl.pallas_call(
        _extract_kernel,
        grid=(NADJ,),
        out_shape=jax.ShapeDtypeStruct((N, HID), f32),
        in_specs=[pl.BlockSpec((ADJ_BLK, N), lambda i: (i, 0)),
                  pl.BlockSpec((N, D_IN), const2),
                  pl.BlockSpec((D_IN, HID), const2),
                  pl.BlockSpec((1, HID), const2),
                  pl.BlockSpec(memory_space=pltpu.SMEM),
                  pl.BlockSpec((1, HID), const2),
                  pl.BlockSpec((1, HID), const2),
                  pl.BlockSpec((HID, ENC), const2),
                  pl.BlockSpec((1, ENC), const2),
                  pl.BlockSpec((ENC, ENC), const2),
                  pl.BlockSpec(memory_space=pltpu.SMEM),
                  pl.BlockSpec((ENC, HID), const2),
                  pl.BlockSpec((1, HID), const2)],
        out_specs=pl.BlockSpec((N, HID), const2),
        scratch_shapes=[pltpu.VMEM((N, HID), f32),
                        pltpu.VMEM((N, D_IN), jnp.bfloat16),
                        pltpu.VMEM((N, HID), f32),
                        pltpu.VMEM((8, HID), f32)],
    )(adj, h, ew1.T, eb1[None, :], eps1, bnw[None, :], bnb[None, :],
      ew2.T, eb2[None, :], wg, bg, wl[:, HID:].T, bl[None, :])

    inp_tm = inputs.transpose(1, 0, 2)

    gconst2 = lambda t: (0, 0)
    gconst3 = lambda t: (0, 0, 0)
    out = pl.pallas_call(
        _gru_kernel,
        grid=(T,),
        out_shape=jax.ShapeDtypeStruct((BN, HID), f32),
        in_specs=[pl.BlockSpec((T, B, N), gconst3),
                  pl.BlockSpec((N, HID), gconst2),
                  pl.BlockSpec((HID, 2 * HID), gconst2),
                  pl.BlockSpec((1, 2 * HID), gconst2),
                  pl.BlockSpec((1, 2 * HID), gconst2),
                  pl.BlockSpec((HID, HID), gconst2),
                  pl.BlockSpec((1, HID), gconst2),
                  pl.BlockSpec((1, HID), gconst2),
                  pl.BlockSpec((HID, HID), gconst2)],
        out_specs=pl.BlockSpec((BN, HID), gconst2),
        scratch_shapes=[pltpu.VMEM((N, 2 * HID), jnp.bfloat16),
                        pltpu.VMEM((N, HID), f32),
                        pltpu.VMEM((N, HID), f32)],
    )(inp_tm, ftc, w1[1:, :], w1[0:1, :], b1[None, :],
      w2[1:, :], w2[0:1, :], b2[None, :], wl[:, :HID].T)

    return out.reshape(B, N, HID)
